# baseline (speedup 1.0000x reference)
import jax
import jax.numpy as jnp
from jax.experimental import pallas as pl
from jax.experimental.pallas import tpu as pltpu

_VOCAB = 8192
_C = 32
_HW = 1024
_B = 32
_NTOK = _B * (_HW + 1)
_BETA = 0.25


def _vq_body(class_ref, patch_ref, w_ref, wt_ref,
             out_class_ref, out_patch_ref, loss_ref, usage_ref,
             wn_ref, used_ref, sumsq_ref):
    b = pl.program_id(0)

    @pl.when(b == 0)
    def _init():
        w = w_ref[...]
        norms = jnp.sqrt(jnp.sum(w * w, axis=1, keepdims=True))
        wn_ref[...] = w / jnp.maximum(norms, 1e-12)
        used_ref[...] = jnp.zeros_like(used_ref)
        sumsq_ref[0] = 0.0

    wn = wn_ref[...]
    wt = wt_ref[...]

    def process(cols):
        norms = jnp.sqrt(jnp.sum(cols * cols, axis=0, keepdims=True))
        coln = cols / jnp.maximum(norms, 1e-12)
        scores = jax.lax.dot_general(
            wn, coln, (((1,), (0,)), ((), ())),
            preferred_element_type=jnp.float32)
        m = jnp.max(scores, axis=0, keepdims=True)
        iot = jax.lax.broadcasted_iota(jnp.int32, scores.shape, 0)
        cand = jnp.where(scores == m, iot, jnp.int32(_VOCAB))
        idx = jnp.min(cand, axis=0, keepdims=True)
        oh = (iot == idx).astype(jnp.float32)
        fhatT = jax.lax.dot_general(
            wt, oh, (((1,), (0,)), ((), ())),
            preferred_element_type=jnp.float32,
            precision=jax.lax.Precision.HIGHEST)
        used_col = jnp.max(oh, axis=1, keepdims=True)
        d = fhatT - cols
        return fhatT, used_col, jnp.sum(d * d)

    @pl.when(b == 0)
    def _class():
        fhatT, used_col, ssq = process(class_ref[...])
        out_class_ref[...] = fhatT
        used_ref[...] = jnp.maximum(used_ref[...], used_col)
        sumsq_ref[0] += ssq

    fhatT, used_col, ssq = process(patch_ref[0])
    out_patch_ref[0] = fhatT
    used_ref[...] = jnp.maximum(used_ref[...], used_col)
    sumsq_ref[0] += ssq

    @pl.when(b == pl.num_programs(0) - 1)
    def _fin():
        loss = (1.0 + _BETA) * sumsq_ref[0] / (_NTOK * _C)
        loss_ref[...] = jnp.full((1, 1), loss, jnp.float32)
        usage = jnp.sum(used_ref[...]) * (100.0 / _VOCAB)
        usage_ref[...] = jnp.full((1, 1), usage, jnp.float32)


def kernel(class_tokens, patch_tokens, emb_weight):
    class_tokens = class_tokens.astype(jnp.float32)
    patch_tokens = patch_tokens.astype(jnp.float32)
    B, L, C = class_tokens.shape
    _, _, H, W = patch_tokens.shape
    classT = class_tokens.reshape(B * L, C).T
    patches = patch_tokens.reshape(B, C, H * W)
    embT = emb_weight.T

    out_shapes = (
        jax.ShapeDtypeStruct((C, B * L), jnp.float32),
        jax.ShapeDtypeStruct((B, C, H * W), jnp.float32),
        jax.ShapeDtypeStruct((1, 1), jnp.float32),
        jax.ShapeDtypeStruct((1, 1), jnp.float32),
    )
    outc, outp, loss, usage = pl.pallas_call(
        _vq_body,
        grid=(B,),
        in_specs=[
            pl.BlockSpec((C, B * L), lambda b: (0, 0)),
            pl.BlockSpec((1, C, H * W), lambda b: (b, 0, 0)),
            pl.BlockSpec((_VOCAB, C), lambda b: (0, 0)),
            pl.BlockSpec((C, _VOCAB), lambda b: (0, 0)),
        ],
        out_specs=[
            pl.BlockSpec((C, B * L), lambda b: (0, 0)),
            pl.BlockSpec((1, C, H * W), lambda b: (b, 0, 0)),
            pl.BlockSpec((1, 1), lambda b: (0, 0)),
            pl.BlockSpec((1, 1), lambda b: (0, 0)),
        ],
        out_shape=out_shapes,
        scratch_shapes=[
            pltpu.VMEM((_VOCAB, C), jnp.float32),
            pltpu.VMEM((_VOCAB, 1), jnp.float32),
            pltpu.SMEM((1,), jnp.float32),
        ],
        compiler_params=pltpu.CompilerParams(
            dimension_semantics=("arbitrary",),
        ),
    )(classT, patches, emb_weight, embT)

    fhat_class = outc.T.reshape(B, L, C)
    fhat_patch = outp.reshape(B, C, H, W)
    return (fhat_class, fhat_patch, loss[0, 0], jnp.float32(0.0), usage[0, 0])


# speedup vs baseline: 2.9112x; 2.9112x over previous
import functools

import jax
import jax.numpy as jnp
from jax.experimental import pallas as pl
from jax.experimental.pallas import tpu as pltpu
from jax.experimental.pallas import tpu_sc as plsc

_VOCAB = 8192
_C = 32
_HW = 1024
_B = 32
_NTOK = _B * (_HW + 1)
_BETA = 0.25


def _argmax_body(classT_ref, patch_ref, w_ref,
                 idx_ref, out_classT_ref, usage_ref, class_ssq_ref,
                 wn_ref, used_ref):
    b = pl.program_id(0)

    @pl.when(b == 0)
    def _init():
        w = w_ref[...]
        norms = jnp.sqrt(jnp.sum(w * w, axis=1, keepdims=True))
        wn_ref[...] = w / jnp.maximum(norms, 1e-12)
        used_ref[...] = jnp.zeros_like(used_ref)

    wn = wn_ref[...]

    def select(cols):
        norms = jnp.sqrt(jnp.sum(cols * cols, axis=0, keepdims=True))
        coln = cols / jnp.maximum(norms, 1e-12)
        scores = jax.lax.dot_general(
            wn, coln, (((1,), (0,)), ((), ())),
            preferred_element_type=jnp.float32)
        m = jnp.max(scores, axis=0, keepdims=True)
        iot = jax.lax.broadcasted_iota(jnp.int32, scores.shape, 0)
        cand = jnp.where(scores == m, iot, jnp.int32(_VOCAB))
        idx = jnp.min(cand, axis=0, keepdims=True)
        oh = (iot == idx).astype(jnp.float32)
        used_col = jnp.max(oh, axis=1, keepdims=True)
        return idx, oh, used_col

    @pl.when(b == 0)
    def _class():
        cl = classT_ref[...]
        _, oh, used_col = select(cl)
        fcT = jax.lax.dot_general(
            w_ref[...], oh, (((0,), (0,)), ((), ())),
            preferred_element_type=jnp.float32,
            precision=jax.lax.Precision.HIGHEST)
        out_classT_ref[...] = fcT
        used_ref[...] = jnp.maximum(used_ref[...], used_col)
        d = fcT - cl
        class_ssq_ref[...] = jnp.full((1, 1), jnp.sum(d * d), jnp.float32)

    idx, _, used_col = select(patch_ref[0])
    idx_ref[...] = idx[None]
    used_ref[...] = jnp.maximum(used_ref[...], used_col)

    @pl.when(b == pl.num_programs(0) - 1)
    def _fin():
        usage = jnp.sum(used_ref[...]) * (100.0 / _VOCAB)
        usage_ref[...] = jnp.full((1, 1), usage, jnp.float32)


def _sc_gather_body(idx_hbm, w_hbm, rows_hbm, idx_v, rows_v, sem):
    c = jax.lax.axis_index("c")
    s = jax.lax.axis_index("s")
    wid = s * 2 + c
    base = wid * _HW
    pltpu.sync_copy(idx_hbm.at[pl.ds(base, _HW)], idx_v)
    copies = []
    for i in range(8):
        copies.append(pltpu.async_copy(
            w_hbm.at[idx_v.at[pl.ds(i * 128, 128)]],
            rows_v.at[pl.ds(i * 128, 128)], sem))
    for cp in copies:
        cp.wait()
    pltpu.sync_copy(rows_v, rows_hbm.at[pl.ds(base, _HW)])


@functools.cache
def _get_sc_gather():
    mesh = plsc.VectorSubcoreMesh(core_axis_name="c", subcore_axis_name="s")
    return pl.kernel(
        _sc_gather_body, mesh=mesh,
        out_type=[
            jax.ShapeDtypeStruct((_B * _HW, _C), jnp.float32),
        ],
        scratch_types=[
            pltpu.VMEM((_HW,), jnp.int32),
            pltpu.VMEM((_HW, _C), jnp.float32),
            pltpu.SemaphoreType.DMA,
        ],
        compiler_params=pltpu.CompilerParams(use_tc_tiling_on_sc=False),
    )


def _fin_body(fhat_ref, patch_ref, cssq_ref, out_ref, loss_ref, ssq_ref):
    b = pl.program_id(0)

    @pl.when(b == 0)
    def _init():
        ssq_ref[0] = cssq_ref[0, 0]

    ft = fhat_ref[0].T
    out_ref[0] = ft
    d = ft - patch_ref[0]
    ssq_ref[0] += jnp.sum(d * d)

    @pl.when(b == pl.num_programs(0) - 1)
    def _fin():
        loss = (1.0 + _BETA) * ssq_ref[0] / (_NTOK * _C)
        loss_ref[...] = jnp.full((1, 1), loss, jnp.float32)


def kernel(class_tokens, patch_tokens, emb_weight):
    class_tokens = class_tokens.astype(jnp.float32)
    patch_tokens = patch_tokens.astype(jnp.float32)
    B, L, C = class_tokens.shape
    _, _, H, W = patch_tokens.shape
    classT = class_tokens.reshape(B * L, C).T
    patches = patch_tokens.reshape(B, C, H * W)

    idx3, fhat_classT, usage, cssq = pl.pallas_call(
        _argmax_body,
        grid=(B,),
        in_specs=[
            pl.BlockSpec((C, B * L), lambda b: (0, 0)),
            pl.BlockSpec((1, C, H * W), lambda b: (b, 0, 0)),
            pl.BlockSpec((_VOCAB, C), lambda b: (0, 0)),
        ],
        out_specs=[
            pl.BlockSpec((1, 1, H * W), lambda b: (b, 0, 0)),
            pl.BlockSpec((C, B * L), lambda b: (0, 0)),
            pl.BlockSpec((1, 1), lambda b: (0, 0)),
            pl.BlockSpec((1, 1), lambda b: (0, 0)),
        ],
        out_shape=(
            jax.ShapeDtypeStruct((B, 1, H * W), jnp.int32),
            jax.ShapeDtypeStruct((C, B * L), jnp.float32),
            jax.ShapeDtypeStruct((1, 1), jnp.float32),
            jax.ShapeDtypeStruct((1, 1), jnp.float32),
        ),
        scratch_shapes=[
            pltpu.VMEM((_VOCAB, C), jnp.float32),
            pltpu.VMEM((_VOCAB, 1), jnp.float32),
        ],
        compiler_params=pltpu.CompilerParams(
            dimension_semantics=("arbitrary",),
        ),
    )(classT, patches, emb_weight)

    idx_flat = idx3.reshape(B * H * W)
    (fhat_rows,) = _get_sc_gather()(idx_flat, emb_weight)

    outp, loss = pl.pallas_call(
        _fin_body,
        grid=(B,),
        in_specs=[
            pl.BlockSpec((1, H * W, C), lambda b: (b, 0, 0)),
            pl.BlockSpec((1, C, H * W), lambda b: (b, 0, 0)),
            pl.BlockSpec((1, 1), lambda b: (0, 0)),
        ],
        out_specs=[
            pl.BlockSpec((1, C, H * W), lambda b: (b, 0, 0)),
            pl.BlockSpec((1, 1), lambda b: (0, 0)),
        ],
        out_shape=(
            jax.ShapeDtypeStruct((B, C, H * W), jnp.float32),
            jax.ShapeDtypeStruct((1, 1), jnp.float32),
        ),
        scratch_shapes=[
            pltpu.SMEM((1,), jnp.float32),
        ],
        compiler_params=pltpu.CompilerParams(
            dimension_semantics=("arbitrary",),
        ),
    )(fhat_rows.reshape(B, H * W, C), patches, cssq)

    fhat_patch = outp.reshape(B, C, H, W)
    return (fhat_classT.T.reshape(B, L, C), fhat_patch, loss[0, 0],
            jnp.float32(0.0), usage[0, 0])


# speedup vs baseline: 3.8595x; 1.3258x over previous
import functools

import jax
import jax.numpy as jnp
from jax.experimental import pallas as pl
from jax.experimental.pallas import tpu as pltpu
from jax.experimental.pallas import tpu_sc as plsc

_VOCAB = 8192
_C = 32
_HW = 1024
_B = 32
_NTOK = _B * (_HW + 1)
_BETA = 0.25
_UW = 16


def _argmax_body(classT_ref, patch_ref, w_ref,
                 idx_ref, out_classT_ref, class_used_ref, class_ssq_ref,
                 wn_ref):
    b = pl.program_id(0)

    @pl.when(b == 0)
    def _init():
        w = w_ref[...]
        norms = jnp.sqrt(jnp.sum(w * w, axis=1, keepdims=True))
        wn_ref[...] = w / jnp.maximum(norms, 1e-12)

    wn = wn_ref[...]

    def select(cols):
        norms = jnp.sqrt(jnp.sum(cols * cols, axis=0, keepdims=True))
        coln = cols / jnp.maximum(norms, 1e-12)
        scores = jax.lax.dot_general(
            wn, coln, (((1,), (0,)), ((), ())),
            preferred_element_type=jnp.float32)
        m = jnp.max(scores, axis=0, keepdims=True)
        iot_i = jax.lax.broadcasted_iota(jnp.int32, scores.shape, 0)
        iot = jax.lax.bitcast_convert_type(
            jnp.bitwise_or(iot_i, jnp.int32(0x4B000000)), jnp.float32)
        cand = jnp.where(scores == m, iot, jnp.float32(8388608.0 + _VOCAB))
        idxf = jnp.min(cand, axis=0, keepdims=True)
        idx = jnp.bitwise_and(
            jax.lax.bitcast_convert_type(idxf, jnp.int32),
            jnp.int32(0x007FFFFF))
        return idx, cand, idxf

    @pl.when(b == 0)
    def _class():
        cl = classT_ref[...]
        _, cand, idxf = select(cl)
        oh = (cand == idxf).astype(jnp.float32)
        fcT = jax.lax.dot_general(
            w_ref[...], oh, (((0,), (0,)), ((), ())),
            preferred_element_type=jnp.float32,
            precision=jax.lax.Precision.HIGHEST)
        out_classT_ref[...] = fcT
        class_used_ref[...] = jnp.max(oh, axis=1, keepdims=True)
        d = fcT - cl
        class_ssq_ref[...] = jnp.full((1, 1), jnp.sum(d * d), jnp.float32)

    idx, _, _ = select(patch_ref[0])
    idx_ref[...] = idx[None]


def _sc_gather_body(idx2_hbm, w_hbm, consts_hbm, rows_hbm, used_hbm,
                    idx2_v, rows_v, zero_v, ones_v, sem):
    c = jax.lax.axis_index("c")
    s = jax.lax.axis_index("s")
    wid = s * 2 + c
    base = wid * _HW
    pltpu.sync_copy(idx2_hbm.at[wid], idx2_v)
    copies = []
    for i in range(8):
        copies.append(pltpu.async_copy(
            w_hbm.at[idx2_v.at[i]],
            rows_v.at[pl.ds(i * 128, 128)], sem))
    for cp in copies:
        cp.wait()
    pltpu.sync_copy(rows_v, rows_hbm.at[pl.ds(base, _HW)])

    pltpu.sync_copy(consts_hbm.at[pl.ds(0, 512)], zero_v)
    pltpu.sync_copy(consts_hbm.at[pl.ds(512, 128)], ones_v)
    pltpu.sync_copy(zero_v, used_hbm.at[c].at[pl.ds(s * 512, 512)])
    plsc.subcore_barrier()
    scats = []
    for i in range(8):
        scats.append(pltpu.async_copy(
            ones_v, used_hbm.at[c].at[idx2_v.at[i]], sem))
    for sc_ in scats:
        sc_.wait()


@functools.cache
def _get_sc_gather():
    mesh = plsc.VectorSubcoreMesh(core_axis_name="c", subcore_axis_name="s")
    return pl.kernel(
        _sc_gather_body, mesh=mesh,
        out_type=[
            jax.ShapeDtypeStruct((_B * _HW, _C), jnp.float32),
            jax.ShapeDtypeStruct((2, _VOCAB, _UW), jnp.float32),
        ],
        scratch_types=[
            pltpu.VMEM((8, 128), jnp.int32),
            pltpu.VMEM((_HW, _C), jnp.float32),
            pltpu.VMEM((512, _UW), jnp.float32),
            pltpu.VMEM((128, _UW), jnp.float32),
            pltpu.SemaphoreType.DMA,
        ],
        compiler_params=pltpu.CompilerParams(use_tc_tiling_on_sc=False),
    )


def _fin_body(fhat_ref, patch_ref, used_sc_ref, class_used_ref, cssq_ref,
              out_ref, loss_ref, usage_ref, ssq_ref):
    b = pl.program_id(0)

    @pl.when(b == 0)
    def _init():
        ssq_ref[0] = cssq_ref[0, 0]

    ft = fhat_ref[0].T
    out_ref[0] = ft
    d = ft - patch_ref[0]
    ssq_ref[0] += jnp.sum(d * d)

    @pl.when(b == pl.num_programs(0) - 1)
    def _fin():
        loss = (1.0 + _BETA) * ssq_ref[0] / (_NTOK * _C)
        loss_ref[...] = jnp.full((1, 1), loss, jnp.float32)
        u = jnp.max(used_sc_ref[...], axis=(0, 2))
        u = jnp.maximum(u[:, None], class_used_ref[...])
        usage_ref[...] = jnp.full((1, 1), jnp.sum(u) * (100.0 / _VOCAB),
                                  jnp.float32)


def kernel(class_tokens, patch_tokens, emb_weight):
    class_tokens = class_tokens.astype(jnp.float32)
    patch_tokens = patch_tokens.astype(jnp.float32)
    B, L, C = class_tokens.shape
    _, _, H, W = patch_tokens.shape
    classT = class_tokens.reshape(B * L, C).T
    patches = patch_tokens.reshape(B, C, H * W)

    idx3, fhat_classT, class_used, cssq = pl.pallas_call(
        _argmax_body,
        grid=(B,),
        in_specs=[
            pl.BlockSpec((C, B * L), lambda b: (0, 0)),
            pl.BlockSpec((1, C, H * W), lambda b: (b, 0, 0)),
            pl.BlockSpec((_VOCAB, C), lambda b: (0, 0)),
        ],
        out_specs=[
            pl.BlockSpec((1, 1, H * W), lambda b: (b, 0, 0)),
            pl.BlockSpec((C, B * L), lambda b: (0, 0)),
            pl.BlockSpec((_VOCAB, 1), lambda b: (0, 0)),
            pl.BlockSpec((1, 1), lambda b: (0, 0)),
        ],
        out_shape=(
            jax.ShapeDtypeStruct((B, 1, H * W), jnp.int32),
            jax.ShapeDtypeStruct((C, B * L), jnp.float32),
            jax.ShapeDtypeStruct((_VOCAB, 1), jnp.float32),
            jax.ShapeDtypeStruct((1, 1), jnp.float32),
        ),
        scratch_shapes=[
            pltpu.VMEM((_VOCAB, C), jnp.float32),
        ],
        compiler_params=pltpu.CompilerParams(
            dimension_semantics=("arbitrary",),
        ),
    )(classT, patches, emb_weight)

    idx2 = idx3.reshape(B, 8, 128)
    consts = jnp.concatenate(
        [jnp.zeros((512, _UW), jnp.float32), jnp.ones((128, _UW), jnp.float32)])
    fhat_rows, used_sc = _get_sc_gather()(idx2, emb_weight, consts)

    outp, loss, usage = pl.pallas_call(
        _fin_body,
        grid=(B,),
        in_specs=[
            pl.BlockSpec((1, H * W, C), lambda b: (b, 0, 0)),
            pl.BlockSpec((1, C, H * W), lambda b: (b, 0, 0)),
            pl.BlockSpec((2, _VOCAB, _UW), lambda b: (0, 0, 0)),
            pl.BlockSpec((_VOCAB, 1), lambda b: (0, 0)),
            pl.BlockSpec((1, 1), lambda b: (0, 0)),
        ],
        out_specs=[
            pl.BlockSpec((1, C, H * W), lambda b: (b, 0, 0)),
            pl.BlockSpec((1, 1), lambda b: (0, 0)),
            pl.BlockSpec((1, 1), lambda b: (0, 0)),
        ],
        out_shape=(
            jax.ShapeDtypeStruct((B, C, H * W), jnp.float32),
            jax.ShapeDtypeStruct((1, 1), jnp.float32),
            jax.ShapeDtypeStruct((1, 1), jnp.float32),
        ),
        scratch_shapes=[
            pltpu.SMEM((1,), jnp.float32),
        ],
        compiler_params=pltpu.CompilerParams(
            dimension_semantics=("arbitrary",),
        ),
    )(fhat_rows.reshape(B, H * W, C), patches, used_sc, class_used, cssq)

    fhat_patch = outp.reshape(B, C, H, W)
    return (fhat_classT.T.reshape(B, L, C), fhat_patch, loss[0, 0],
            jnp.float32(0.0), usage[0, 0])
